# Initial kernel scaffold; baseline (speedup 1.0000x reference)
import jax
import jax.numpy as jnp
from jax.experimental import pallas as pl


def kernel(feat, segment_ids, num_graphs):
    raise NotImplementedError("write your pallas kernel here")


# speedup vs baseline: 2.6332x; 2.6332x over previous
import functools
import jax
import jax.numpy as jnp
from jax import lax
from jax.experimental import pallas as pl
from jax.experimental.pallas import tpu as pltpu
from jax.experimental.pallas import tpu_sc as plsc

E = 320000
D = 128
G = 256

NC = 2
NS = 16
DC = D // NC
EPS = E // NS
CH = 80
NCHUNK = EPS // CH


def _sc_body(feat_hbm, seg_hbm, out_hbm, feat_v, idx_v, acc_sh):
    c = lax.axis_index("c")
    s = lax.axis_index("s")
    col0 = c * DC

    zero = jnp.zeros((16,), jnp.float32)
    for r in range(16):
        for cc in range(DC // 16):
            feat_v[r, pl.ds(cc * 16, 16)] = zero
    pltpu.sync_copy(feat_v.at[pl.ds(0, 16)], acc_sh.at[pl.ds(s * 16, 16)])
    plsc.subcore_barrier()

    base = s * EPS

    def chunk_step(i, carry):
        off = base + i * CH
        pltpu.sync_copy(seg_hbm.at[pl.ds(off, CH)], idx_v)
        pltpu.sync_copy(feat_hbm.at[pl.ds(off, CH), pl.ds(col0, DC)], feat_v)
        pltpu.sync_copy(feat_v, acc_sh.at[idx_v], add=True)
        return carry

    lax.fori_loop(0, NCHUNK, chunk_step, 0)
    plsc.subcore_barrier()

    pltpu.sync_copy(acc_sh.at[pl.ds(s * 16, 16)],
                    out_hbm.at[pl.ds(s * 16, 16), pl.ds(col0, DC)])


@jax.jit
def _sum_pool(feat, segment_ids):
    mesh = plsc.VectorSubcoreMesh(core_axis_name="c", subcore_axis_name="s")
    f = pl.kernel(
        _sc_body,
        out_type=jax.ShapeDtypeStruct((G, D), jnp.float32),
        mesh=mesh,
        scratch_types=[
            pltpu.VMEM((CH, DC), jnp.float32),
            pltpu.VMEM((CH,), jnp.int32),
            pltpu.VMEM_SHARED((G, DC), jnp.float32),
        ],
        compiler_params=pltpu.CompilerParams(use_tc_tiling_on_sc=False),
        name="segment_sum_pool_sc",
    )
    return f(feat, segment_ids)


def kernel(feat, segment_ids, num_graphs):
    num_graphs = jnp.asarray(num_graphs, dtype=jnp.int32)
    segment_ids = segment_ids + (num_graphs - jnp.int32(G))
    return _sum_pool(feat, segment_ids)


# speedup vs baseline: 4.7491x; 1.8035x over previous
import functools
import jax
import jax.numpy as jnp
from jax import lax
from jax.experimental import pallas as pl
from jax.experimental.pallas import tpu as pltpu
from jax.experimental.pallas import tpu_sc as plsc

E = 320000
D = 128
G = 256

NC = 2
NS = 16
DC = D // NC
EPS = E // NS
CH = 80
NCHUNK = EPS // CH


def _sc_body(feat_hbm, seg2_hbm, out_hbm, idx_v, fa, fb, acc_sh, sa, sb):
    c = lax.axis_index("c")
    s = lax.axis_index("s")
    col0 = c * DC
    base = s * EPS

    zero = jnp.zeros((16,), jnp.float32)
    for r in range(16):
        for cc in range(DC // 16):
            fa[r, pl.ds(cc * 16, 16)] = zero
    pltpu.sync_copy(fa.at[pl.ds(0, 16)], acc_sh.at[pl.ds(s * 16, 16)])

    pltpu.sync_copy(seg2_hbm.at[pl.ds(s * NCHUNK, NCHUNK)], idx_v)
    plsc.subcore_barrier()

    bufs = (fa, fb)
    sems = (sa, sb)

    def feat_src(chunk):
        return feat_hbm.at[pl.ds(base + chunk * CH, CH), pl.ds(col0, DC)]

    pltpu.async_copy(feat_src(0), fa, sa)
    pltpu.async_copy(feat_src(1), fb, sb)

    def pair_step(i, carry):
        for b in range(2):
            chunk = 2 * i + b
            buf, sem = bufs[b], sems[b]
            pltpu.make_async_copy(feat_src(chunk), buf, sem).wait()
            pltpu.sync_copy(buf, acc_sh.at[idx_v.at[chunk]], add=True)

            @pl.when(chunk + 2 < NCHUNK)
            def _():
                pltpu.async_copy(feat_src(chunk + 2), buf, sem)

        return carry

    lax.fori_loop(0, NCHUNK // 2, pair_step, 0)
    plsc.subcore_barrier()

    pltpu.sync_copy(acc_sh.at[pl.ds(s * 16, 16)],
                    out_hbm.at[pl.ds(s * 16, 16), pl.ds(col0, DC)])


@jax.jit
def _sum_pool(feat, segment_ids):
    mesh = plsc.VectorSubcoreMesh(core_axis_name="c", subcore_axis_name="s")
    f = pl.kernel(
        _sc_body,
        out_type=jax.ShapeDtypeStruct((G, D), jnp.float32),
        mesh=mesh,
        scratch_types=[
            pltpu.VMEM((NS * NCHUNK // NS, CH), jnp.int32),
            pltpu.VMEM((CH, DC), jnp.float32),
            pltpu.VMEM((CH, DC), jnp.float32),
            pltpu.VMEM_SHARED((G, DC), jnp.float32),
            pltpu.SemaphoreType.DMA,
            pltpu.SemaphoreType.DMA,
        ],
        compiler_params=pltpu.CompilerParams(use_tc_tiling_on_sc=False),
        name="segment_sum_pool_sc",
    )
    return f(feat, segment_ids.reshape(E // CH, CH))


def kernel(feat, segment_ids, num_graphs):
    num_graphs = jnp.asarray(num_graphs, dtype=jnp.int32)
    segment_ids = segment_ids + (num_graphs - jnp.int32(G))
    return _sum_pool(feat, segment_ids)


# speedup vs baseline: 4.8643x; 1.0243x over previous
import functools
import jax
import jax.numpy as jnp
from jax import lax
from jax.experimental import pallas as pl
from jax.experimental.pallas import tpu as pltpu
from jax.experimental.pallas import tpu_sc as plsc

E = 320000
D = 128
G = 256

NC = 2
NS = 16
DC = D // NC
EPS = E // NS
CH = 80
NCHUNK = EPS // CH
NBUF = 5
LOOKAHEAD = 3


def _sc_body(feat_hbm, seg2_hbm, out_hbm, idx_v,
             f0, f1, f2, f3, f4, acc_sh,
             l0, l1, l2, l3, l4, s0, s1, s2, s3, s4):
    c = lax.axis_index("c")
    s = lax.axis_index("s")
    col0 = c * DC
    base = s * EPS
    bufs = (f0, f1, f2, f3, f4)
    lsem = (l0, l1, l2, l3, l4)
    ssem = (s0, s1, s2, s3, s4)

    zero = jnp.zeros((16,), jnp.float32)
    for r in range(16):
        for cc in range(DC // 16):
            f0[r, pl.ds(cc * 16, 16)] = zero
    pltpu.sync_copy(f0.at[pl.ds(0, 16)], acc_sh.at[pl.ds(s * 16, 16)])

    pltpu.sync_copy(seg2_hbm.at[pl.ds(s * NCHUNK, NCHUNK)], idx_v)

    def feat_src(chunk):
        return feat_hbm.at[pl.ds(base + chunk * CH, CH), pl.ds(col0, DC)]

    for b in range(NBUF):
        pltpu.async_copy(feat_src(b), bufs[b], lsem[b])
    plsc.subcore_barrier()

    def quint_step(i, carry):
        for b in range(NBUF):
            chunk = NBUF * i + b
            pltpu.make_async_copy(feat_src(chunk), bufs[b], lsem[b]).wait()
            pltpu.async_copy(bufs[b], acc_sh.at[idx_v.at[chunk]], ssem[b],
                             add=True)

            t = chunk + NBUF - (NBUF - LOOKAHEAD)
            bt = (b + LOOKAHEAD) % NBUF

            @pl.when((t >= NBUF) & (t < NCHUNK))
            def _():
                pltpu.make_async_copy(
                    bufs[bt], acc_sh.at[idx_v.at[0]], ssem[bt]).wait()
                pltpu.async_copy(feat_src(t), bufs[bt], lsem[bt])

        return carry

    lax.fori_loop(0, NCHUNK // NBUF, quint_step, 0)

    for b in range(NBUF):
        pltpu.make_async_copy(bufs[b], acc_sh.at[idx_v.at[0]], ssem[b]).wait()
    plsc.subcore_barrier()

    pltpu.sync_copy(acc_sh.at[pl.ds(s * 16, 16)],
                    out_hbm.at[pl.ds(s * 16, 16), pl.ds(col0, DC)])


@jax.jit
def _sum_pool(feat, segment_ids):
    mesh = plsc.VectorSubcoreMesh(core_axis_name="c", subcore_axis_name="s")
    f = pl.kernel(
        _sc_body,
        out_type=jax.ShapeDtypeStruct((G, D), jnp.float32),
        mesh=mesh,
        scratch_types=(
            [pltpu.VMEM((NCHUNK, CH), jnp.int32)]
            + [pltpu.VMEM((CH, DC), jnp.float32)] * NBUF
            + [pltpu.VMEM_SHARED((G, DC), jnp.float32)]
            + [pltpu.SemaphoreType.DMA] * (2 * NBUF)
        ),
        compiler_params=pltpu.CompilerParams(use_tc_tiling_on_sc=False),
        name="segment_sum_pool_sc",
    )
    return f(feat, segment_ids.reshape(E // CH, CH))


def kernel(feat, segment_ids, num_graphs):
    num_graphs = jnp.asarray(num_graphs, dtype=jnp.int32)
    segment_ids = segment_ids + (num_graphs - jnp.int32(G))
    return _sum_pool(feat, segment_ids)


# speedup vs baseline: 8.7265x; 1.7940x over previous
import functools
import jax
import jax.numpy as jnp
from jax import lax
from jax.experimental import pallas as pl
from jax.experimental.pallas import tpu as pltpu
from jax.experimental.pallas import tpu_sc as plsc

E = 320000
D = 128
G = 256

NC = 2
NS = 16
DC = D // NC
NG = DC // 16
EPS = E // NS
CH = 80
NCHUNK = EPS // CH
NBUF = 5
LOOKAHEAD = 3
RUNROLL = 8


def _sc_body(feat_hbm, seg2_hbm, fidx_hbm, out_hbm,
             idx_v, fidx_v, acc_l,
             f0, f1, f2, f3, f4, acc_sh,
             l0, l1, l2, l3, l4, ssem):
    c = lax.axis_index("c")
    s = lax.axis_index("s")
    col0 = c * DC
    base = s * EPS
    bufs = (f0, f1, f2, f3, f4)
    lsem = (l0, l1, l2, l3, l4)

    zero = jnp.zeros((16,), jnp.float32)

    def zrow(r, carry):
        for g in range(NG):
            acc_l[r, pl.ds(g * 16, 16)] = zero
        return carry

    lax.fori_loop(0, G, zrow, 0)
    pltpu.sync_copy(acc_l.at[pl.ds(0, 16)], acc_sh.at[pl.ds(s * 16, 16)])

    pltpu.sync_copy(seg2_hbm.at[pl.ds(s * NCHUNK, NCHUNK)], idx_v)
    pltpu.sync_copy(fidx_hbm, fidx_v)

    def feat_src(chunk):
        return feat_hbm.at[pl.ds(base + chunk * CH, CH), pl.ds(col0, DC)]

    for b in range(NBUF):
        pltpu.async_copy(feat_src(b), bufs[b], lsem[b])
    plsc.subcore_barrier()

    def quint_step(i, carry):
        for b in range(NBUF):
            chunk = NBUF * i + b
            buf = bufs[b]
            pltpu.make_async_copy(feat_src(chunk), buf, lsem[b]).wait()

            ivs = [idx_v[chunk, pl.ds(k * 16, 16)] for k in range(CH // 16)]
            vmx = functools.reduce(jnp.maximum, ivs)
            vmn = functools.reduce(jnp.minimum, ivs)
            mx = jnp.max(vmx)
            mn = jnp.min(vmn)

            @pl.when(mx == mn)
            def _():
                def srow(j, acc):
                    accs = list(acc)
                    for r in range(RUNROLL):
                        row = j * RUNROLL + r
                        for g in range(NG):
                            accs[g] = accs[g] + buf[row, pl.ds(g * 16, 16)]
                    return tuple(accs)

                sums = lax.fori_loop(0, CH // RUNROLL, srow,
                                     tuple(zero for _ in range(NG)))
                for g in range(NG):
                    plsc.addupdate(acc_l.at[mx, pl.ds(g * 16, 16)], sums[g])

            @pl.when(mx != mn)
            def _():
                pltpu.sync_copy(buf, acc_sh.at[idx_v.at[chunk]], add=True)

            t = chunk + LOOKAHEAD
            bt = (b + LOOKAHEAD) % NBUF

            @pl.when((t >= NBUF) & (t < NCHUNK))
            def _():
                pltpu.async_copy(feat_src(t), bufs[bt], lsem[bt])

        return carry

    lax.fori_loop(0, NCHUNK // NBUF, quint_step, 0)

    pltpu.async_copy(acc_l.at[pl.ds(0, 128)], acc_sh.at[fidx_v.at[0]], ssem,
                     add=True)
    pltpu.async_copy(acc_l.at[pl.ds(128, 128)], acc_sh.at[fidx_v.at[1]], ssem,
                     add=True)
    pltpu.make_async_copy(acc_l.at[pl.ds(0, 128)], acc_sh.at[fidx_v.at[0]],
                          ssem).wait()
    pltpu.make_async_copy(acc_l.at[pl.ds(128, 128)], acc_sh.at[fidx_v.at[1]],
                          ssem).wait()
    plsc.subcore_barrier()

    pltpu.sync_copy(acc_sh.at[pl.ds(s * 16, 16)],
                    out_hbm.at[pl.ds(s * 16, 16), pl.ds(col0, DC)])


@jax.jit
def _sum_pool(feat, segment_ids):
    mesh = plsc.VectorSubcoreMesh(core_axis_name="c", subcore_axis_name="s")
    f = pl.kernel(
        _sc_body,
        out_type=jax.ShapeDtypeStruct((G, D), jnp.float32),
        mesh=mesh,
        scratch_types=(
            [pltpu.VMEM((NCHUNK, CH), jnp.int32),
             pltpu.VMEM((2, 128), jnp.int32),
             pltpu.VMEM((G, DC), jnp.float32)]
            + [pltpu.VMEM((CH, DC), jnp.float32)] * NBUF
            + [pltpu.VMEM_SHARED((G, DC), jnp.float32)]
            + [pltpu.SemaphoreType.DMA] * (NBUF + 1)
        ),
        compiler_params=pltpu.CompilerParams(use_tc_tiling_on_sc=False,
                                             needs_layout_passes=False),
        name="segment_sum_pool_sc",
    )
    fidx = jnp.arange(G, dtype=jnp.int32).reshape(2, 128)
    return f(feat, segment_ids.reshape(E // CH, CH), fidx)


def kernel(feat, segment_ids, num_graphs):
    num_graphs = jnp.asarray(num_graphs, dtype=jnp.int32)
    segment_ids = segment_ids + (num_graphs - jnp.int32(G))
    return _sum_pool(feat, segment_ids)
